# baseline (speedup 1.0000x reference)
import functools

import jax
import jax.numpy as jnp
from jax import lax
from jax.experimental import pallas as pl
from jax.experimental.pallas import tpu as pltpu
from jax.experimental.pallas import tpu_sc as plsc

_ITEM_NUM = 1000000
_EMBED_DIM = 64
_ENCODER_SIZE = 128
_CHUNK = 128


def _make_sc_gather(total_rows: int):
    info = plsc.get_sparse_core_info()
    nw = info.num_cores * info.num_subcores
    num_chunks = total_rows // _CHUNK
    cpw = num_chunks // nw
    assert cpw * nw * _CHUNK == total_rows

    mesh = plsc.VectorSubcoreMesh(core_axis_name="c", subcore_axis_name="s")

    @functools.partial(
        pl.kernel,
        out_type=jax.ShapeDtypeStruct((total_rows, _EMBED_DIM), jnp.float32),
        mesh=mesh,
        scratch_types=[
            pltpu.VMEM((cpw, _CHUNK), jnp.int32),
            pltpu.VMEM((_CHUNK, _EMBED_DIM), jnp.float32),
            pltpu.SemaphoreType.DMA,
        ],
        compiler_params=pltpu.CompilerParams(use_tc_tiling_on_sc=False),
    )
    def sc_gather(idx_hbm, table_hbm, out_hbm, idx_v, rows_v, gsem):
        wid = lax.axis_index("s") * info.num_cores + lax.axis_index("c")
        chunk0 = wid * cpw
        pltpu.sync_copy(idx_hbm.at[pl.ds(chunk0, cpw)], idx_v)

        def body(j, _):
            pltpu.async_copy(table_hbm.at[idx_v.at[j]], rows_v, gsem).wait()
            pltpu.sync_copy(
                rows_v, out_hbm.at[pl.ds((chunk0 + j) * _CHUNK, _CHUNK)]
            )
            return ()

        lax.fori_loop(0, cpw, body, ())

    return sc_gather


def _mm_body(x_ref, ids_ref, wt_ref, b_ref, o_ref):
    x = x_ref[...]
    acc = jnp.dot(x, wt_ref[...], preferred_element_type=jnp.float32)
    mask = (ids_ref[...] != 0).astype(jnp.float32)
    o_ref[...] = acc * mask + b_ref[...]


def _make_tc_matmul(total_rows: int, rows_per_block: int):
    grid = (total_rows // rows_per_block,)
    return pl.pallas_call(
        _mm_body,
        grid=grid,
        in_specs=[
            pl.BlockSpec((rows_per_block, _EMBED_DIM), lambda i: (i, 0)),
            pl.BlockSpec((rows_per_block, 1), lambda i: (i, 0)),
            pl.BlockSpec((_EMBED_DIM, _ENCODER_SIZE), lambda i: (0, 0)),
            pl.BlockSpec((1, _ENCODER_SIZE), lambda i: (0, 0)),
        ],
        out_specs=pl.BlockSpec((rows_per_block, _ENCODER_SIZE), lambda i: (i, 0)),
        out_shape=jax.ShapeDtypeStruct(
            (total_rows, _ENCODER_SIZE), jnp.float32
        ),
    )


def kernel(news_ids, table, W, b):
    batch, num_docs = news_ids.shape
    total_rows = batch * num_docs
    ids_flat = news_ids.reshape(-1).astype(jnp.int32)

    vecs = _make_sc_gather(total_rows)(
        ids_flat.reshape(total_rows // _CHUNK, _CHUNK), table
    )
    out = _make_tc_matmul(total_rows, 1024)(
        vecs, ids_flat.reshape(total_rows, 1), W.T, b.reshape(1, _ENCODER_SIZE)
    )
    return out.reshape(batch, num_docs, _ENCODER_SIZE)


# speedup vs baseline: 1.6757x; 1.6757x over previous
import functools

import jax
import jax.numpy as jnp
from jax import lax
from jax.experimental import pallas as pl
from jax.experimental.pallas import tpu as pltpu
from jax.experimental.pallas import tpu_sc as plsc

_ENCODER_SIZE = 128
_PROJ_BLOCK = 8000


def _project_body(t_ref, wt_ref, b_ref, o_ref):
    acc = jnp.dot(t_ref[...], wt_ref[...], preferred_element_type=jnp.float32)
    o_ref[...] = acc + b_ref[...]

    @pl.when(pl.program_id(0) == 0)
    def _():
        o_ref[0:1, :] = b_ref[...]


def _make_project(item_num: int, embed_dim: int):
    grid = (item_num // _PROJ_BLOCK,)
    return pl.pallas_call(
        _project_body,
        grid=grid,
        in_specs=[
            pl.BlockSpec((_PROJ_BLOCK, embed_dim), lambda i: (i, 0)),
            pl.BlockSpec((embed_dim, _ENCODER_SIZE), lambda i: (0, 0)),
            pl.BlockSpec((1, _ENCODER_SIZE), lambda i: (0, 0)),
        ],
        out_specs=pl.BlockSpec((_PROJ_BLOCK, _ENCODER_SIZE), lambda i: (i, 0)),
        out_shape=jax.ShapeDtypeStruct((item_num, _ENCODER_SIZE), jnp.float32),
    )


def _make_sc_gather(batch: int, num_docs: int):
    info = plsc.get_sparse_core_info()
    nw = info.num_cores * info.num_subcores
    bpw = batch // nw
    assert bpw * nw == batch

    mesh = plsc.VectorSubcoreMesh(core_axis_name="c", subcore_axis_name="s")

    @functools.partial(
        pl.kernel,
        out_type=jax.ShapeDtypeStruct(
            (batch, num_docs, _ENCODER_SIZE), jnp.float32
        ),
        mesh=mesh,
        scratch_types=[
            pltpu.VMEM((bpw, num_docs), jnp.int32),
            pltpu.VMEM((num_docs, _ENCODER_SIZE), jnp.float32),
            pltpu.SemaphoreType.DMA,
        ],
    )
    def sc_gather(ids_hbm, ptb_hbm, out_hbm, ids_v, rows_v, gsem):
        wid = lax.axis_index("s") * info.num_cores + lax.axis_index("c")
        base = wid * bpw
        pltpu.sync_copy(ids_hbm.at[pl.ds(base, bpw)], ids_v)

        def body(j, _):
            pltpu.async_copy(ptb_hbm.at[ids_v.at[j]], rows_v, gsem).wait()
            pltpu.sync_copy(rows_v, out_hbm.at[base + j])
            return ()

        lax.fori_loop(0, bpw, body, ())

    return sc_gather


def kernel(news_ids, table, W, b):
    batch, num_docs = news_ids.shape
    item_num, embed_dim = table.shape

    ptb = _make_project(item_num, embed_dim)(
        table, W.T, b.reshape(1, _ENCODER_SIZE)
    )
    return _make_sc_gather(batch, num_docs)(news_ids.astype(jnp.int32), ptb)


# speedup vs baseline: 1.9952x; 1.1907x over previous
import functools

import jax
import jax.numpy as jnp
from jax import lax
from jax.experimental import pallas as pl
from jax.experimental.pallas import tpu as pltpu
from jax.experimental.pallas import tpu_sc as plsc

_ENCODER_SIZE = 128
_PROJ_BLOCK = 8000


def _project_body(t_ref, wt_ref, b_ref, o_ref):
    acc = jnp.dot(t_ref[...], wt_ref[...], preferred_element_type=jnp.float32)
    o_ref[...] = acc + b_ref[...]

    @pl.when(pl.program_id(0) == 0)
    def _():
        o_ref[0:1, :] = b_ref[...]


def _make_project(item_num: int, embed_dim: int):
    grid = (item_num // _PROJ_BLOCK,)
    return pl.pallas_call(
        _project_body,
        grid=grid,
        in_specs=[
            pl.BlockSpec((_PROJ_BLOCK, embed_dim), lambda i: (i, 0)),
            pl.BlockSpec((embed_dim, _ENCODER_SIZE), lambda i: (0, 0)),
            pl.BlockSpec((1, _ENCODER_SIZE), lambda i: (0, 0)),
        ],
        out_specs=pl.BlockSpec((_PROJ_BLOCK, _ENCODER_SIZE), lambda i: (i, 0)),
        out_shape=jax.ShapeDtypeStruct((item_num, _ENCODER_SIZE), jnp.float32),
    )


def _make_sc_gather(batch: int, num_docs: int):
    info = plsc.get_sparse_core_info()
    nw = info.num_cores * info.num_subcores
    bpw = batch // nw
    assert bpw * nw == batch

    mesh = plsc.VectorSubcoreMesh(core_axis_name="c", subcore_axis_name="s")

    @functools.partial(
        pl.kernel,
        out_type=jax.ShapeDtypeStruct(
            (batch, num_docs, _ENCODER_SIZE), jnp.float32
        ),
        mesh=mesh,
        scratch_types=[
            pltpu.VMEM((bpw, num_docs), jnp.int32),
            pltpu.VMEM((num_docs, _ENCODER_SIZE), jnp.float32),
            pltpu.VMEM((num_docs, _ENCODER_SIZE), jnp.float32),
            pltpu.SemaphoreType.DMA,
            pltpu.SemaphoreType.DMA,
            pltpu.SemaphoreType.DMA,
            pltpu.SemaphoreType.DMA,
        ],
    )
    def sc_gather(
        ids_hbm, ptb_hbm, out_hbm, ids_v, rows0, rows1, g0, g1, o0, o1
    ):
        wid = lax.axis_index("s") * info.num_cores + lax.axis_index("c")
        base = wid * bpw
        pltpu.sync_copy(ids_hbm.at[pl.ds(base, bpw)], ids_v)

        def gather(j, buf, sem):
            pltpu.async_copy(ptb_hbm.at[ids_v.at[j]], buf, sem)

        def gather_wait(buf, sem):
            pltpu.make_async_copy(ptb_hbm.at[ids_v.at[0]], buf, sem).wait()

        def put(j, buf, sem):
            pltpu.async_copy(buf, out_hbm.at[base + j], sem)

        def put_wait(j, buf, sem):
            pltpu.make_async_copy(buf, out_hbm.at[base + j], sem).wait()

        gather(0, rows0, g0)
        gather(1, rows1, g1)

        def pair(k, _):
            j = 2 * k
            gather_wait(rows0, g0)
            put(j, rows0, o0)
            gather_wait(rows1, g1)
            put(j + 1, rows1, o1)
            put_wait(j, rows0, o0)

            @pl.when(j + 2 < bpw)
            def _():
                gather(j + 2, rows0, g0)

            put_wait(j + 1, rows1, o1)

            @pl.when(j + 3 < bpw)
            def _():
                gather(j + 3, rows1, g1)

            return ()

        lax.fori_loop(0, bpw // 2, pair, ())

    return sc_gather


def kernel(news_ids, table, W, b):
    batch, num_docs = news_ids.shape
    item_num, embed_dim = table.shape

    ptb = _make_project(item_num, embed_dim)(
        table, W.T, b.reshape(1, _ENCODER_SIZE)
    )
    return _make_sc_gather(batch, num_docs)(news_ids.astype(jnp.int32), ptb)
